# baseline (speedup 1.0000x reference)
import functools

import jax
import jax.numpy as jnp
from jax import lax
from jax.experimental import pallas as pl
from jax.experimental.pallas import tpu as pltpu
from jax.experimental.pallas import tpu_sc as plsc

LANE = 128
HBLK = 8


def kernel(data, table):
    nb, hist = data.shape
    vocab, emb = table.shape
    tbl1 = table.T.reshape(emb * vocab)
    data_t = data.T

    info = plsc.get_sparse_core_info()
    ncores, nsub = info.num_cores, info.num_subcores
    cpc = emb // ncores
    bpt = nb // nsub
    nspan = 10
    span = vocab // nspan
    nblk = hist // HBLK
    jper = bpt // LANE

    mesh = plsc.VectorSubcoreMesh(core_axis_name="c", subcore_axis_name="s")

    @functools.partial(
        pl.kernel,
        mesh=mesh,
        out_type=jax.ShapeDtypeStruct((hist, emb * nb), jnp.float32),
        scratch_types=[
            pltpu.VMEM((hist, bpt), jnp.int32),
            pltpu.VMEM((2, HBLK, bpt), jnp.float32),
            pltpu.VMEM((4000,), jnp.float32),
            pltpu.VMEM((4000,), jnp.float32),
            pltpu.VMEM_SHARED((vocab,), jnp.float32),
            pltpu.SemaphoreType.DMA,
            pltpu.SemaphoreType.DMA,
            pltpu.SemaphoreType.DMA,
            pltpu.SemaphoreType.DMA((2,)),
        ],
    )
    def run(idx_hbm, tbl_hbm, out_hbm, idx_v, buf, stage_a, stage_b, chan,
            isem, osem, gsem, ssem):
        stages = (stage_a, stage_b)
        ci = lax.axis_index("c")
        si = lax.axis_index("s")
        b0 = si * bpt
        pltpu.sync_copy(idx_hbm.at[pl.ds(0, hist), pl.ds(b0, bpt)], idx_v)

        def chan_body(k, carry):
            c = ci * cpc + k

            @pl.when(si < nspan)
            def _():
                piece = 4000
                npc = span // piece

                def fire_in(p):
                    pltpu.async_copy(
                        tbl_hbm.at[pl.ds(c * vocab + si * span + p * piece, piece)],
                        stages[p % 2],
                        isem,
                    )

                def fire_out(p):
                    pltpu.async_copy(
                        stages[p % 2],
                        chan.at[pl.ds(si * span + p * piece, piece)],
                        osem,
                    )

                def wait_in():
                    pltpu.make_async_copy(
                        tbl_hbm.at[pl.ds(0, piece)], stages[0], isem
                    ).wait()

                def wait_out():
                    pltpu.make_async_copy(
                        stages[0], chan.at[pl.ds(0, piece)], osem
                    ).wait()

                fire_in(0)
                for p in range(npc):
                    wait_in()
                    if p >= 1:
                        wait_out()
                    if p + 1 < npc:
                        fire_in(p + 1)
                    fire_out(p)
                wait_out()

            plsc.subcore_barrier()
            col0 = c * nb + b0

            for blk in range(nblk):
                slot = blk % 2
                h0 = blk * HBLK

                @pl.when(k * nblk + blk >= 2)
                def _(slot=slot):
                    pltpu.make_async_copy(
                        buf.at[0],
                        out_hbm.at[pl.ds(0, HBLK), pl.ds(0, bpt)],
                        ssem.at[slot],
                    ).wait()

                def gfire(h, c2, slot=slot, h0=h0):
                    for j in range(jper):
                        pltpu.async_copy(
                            chan.at[idx_v.at[h0 + h, pl.ds(j * LANE, LANE)]],
                            buf.at[slot].at[h, pl.ds(j * LANE, LANE)],
                            gsem,
                        )
                    return c2

                lax.fori_loop(0, HBLK, gfire, 0)

                def gdrain(h, c2):
                    for j in range(jper):
                        pltpu.make_async_copy(
                            chan.at[idx_v.at[0, pl.ds(0, LANE)]],
                            buf.at[0].at[0, pl.ds(0, LANE)],
                            gsem,
                        ).wait()
                    return c2

                lax.fori_loop(0, HBLK, gdrain, 0)
                pltpu.async_copy(
                    buf.at[slot],
                    out_hbm.at[pl.ds(h0, HBLK), pl.ds(col0, bpt)],
                    ssem.at[slot],
                )
            plsc.subcore_barrier()
            return carry

        lax.fori_loop(0, cpc, chan_body, 0)
        for slot in range(2):
            pltpu.make_async_copy(
                buf.at[0],
                out_hbm.at[pl.ds(0, HBLK), pl.ds(0, bpt)],
                ssem.at[slot],
            ).wait()

    out2d = run(data_t, tbl1)
    return out2d.reshape(hist, emb, nb).transpose(2, 0, 1)
